# baseline (speedup 1.0000x reference)
import functools

import jax
import jax.numpy as jnp
from jax import lax
from jax.experimental import pallas as pl
from jax.experimental.pallas import tpu as pltpu
from jax.experimental.pallas import tpu_sc as plsc

_MAXD = 2048
_N = 4096
_T = 2 * _MAXD - 1
_EREV = 8192
_NSHIFT = 16
_NW = 32
_ROWS = _N // _NW
_CHUNK = _ROWS - _NSHIFT + _N


def _body(erevx_hbm, out_hbm, chunkx, ldsem, stsem):
    wid = lax.axis_index("c") * 16 + lax.axis_index("s")
    i0 = wid * _ROWS
    start_min = (_N - _ROWS) - i0

    loads = [
        pltpu.async_copy(
            erevx_hbm.at[pl.ds(_EREV * c + start_min, _CHUNK)],
            chunkx.at[pl.ds(_CHUNK * c, _CHUNK)],
            ldsem,
        )
        for c in range(_NSHIFT)
    ]
    for h in loads:
        h.wait()

    batches = []
    for c in range(_NSHIFT):
        batch = [
            pltpu.async_copy(
                chunkx.at[pl.ds(_CHUNK * c + 16 * t, _N)],
                out_hbm.at[i0 + _ROWS - 1 - c - 16 * t],
                stsem,
            )
            for t in range(_ROWS // _NSHIFT)
        ]
        batches.append(batch)
        if c >= 1:
            for h in batches[c - 1]:
                h.wait()
    for h in batches[-1]:
        h.wait()


def _toeplitz_rows(erevx):
    mesh = plsc.VectorSubcoreMesh(core_axis_name="c", subcore_axis_name="s")
    f = functools.partial(
        pl.kernel,
        mesh=mesh,
        out_type=jax.ShapeDtypeStruct((_N, _N), jnp.float32),
        scratch_types=[
            pltpu.VMEM((_NSHIFT * _CHUNK,), jnp.float32),
            pltpu.SemaphoreType.DMA,
            pltpu.SemaphoreType.DMA,
        ],
        compiler_params=pltpu.CompilerParams(use_tc_tiling_on_sc=False),
    )(_body)
    return f(erevx)


def kernel(bias_table, query_len, cond_len):
    d = jnp.asarray(query_len, jnp.int32) - jnp.asarray(cond_len, jnp.int32)
    base = jnp.pad(jnp.flip(bias_table), (_N, _N), mode="edge")
    dc = jnp.clip(d, -2000, 2000)
    big = lax.dynamic_slice(base, (_MAXD - dc,), (_EREV + _NSHIFT,))
    erevx = jnp.concatenate([big[c : c + _EREV] for c in range(_NSHIFT)])
    return _toeplitz_rows(erevx)


# speedup vs baseline: 1.1869x; 1.1869x over previous
import functools

import jax
import jax.numpy as jnp
from jax import lax
from jax.experimental import pallas as pl
from jax.experimental.pallas import tpu as pltpu
from jax.experimental.pallas import tpu_sc as plsc

_MAXD = 2048
_N = 4096
_T = 2 * _MAXD - 1
_EREV = 8192
_NW = 32
_ROWS = _N // _NW
_BLK = 8
_NBLK = _ROWS // _BLK
_NBUF = 3


def _body(g_hbm, idx_hbm, out_hbm, rowbuf, idxv, ldsem, stsem):
    wid = lax.axis_index("c") * 16 + lax.axis_index("s")
    i0 = wid * _ROWS
    start_min = (_N - _ROWS) - i0

    pltpu.sync_copy(idx_hbm.at[wid], idxv)

    lds, sts = [], []

    def _scatter(b):
        return pltpu.async_copy(
            rowbuf.at[b % _NBUF], out_hbm.at[idxv.at[b]], stsem
        )

    for b in range(_NBLK):
        if b >= _NBUF:
            sts[b - _NBUF].wait()
        lds.append(
            pltpu.async_copy(
                g_hbm.at[b, :, pl.ds(start_min, _N)],
                rowbuf.at[b % _NBUF],
                ldsem,
            )
        )
        if b >= 1:
            lds[b - 1].wait()
            sts.append(_scatter(b - 1))
    lds[_NBLK - 1].wait()
    sts.append(_scatter(_NBLK - 1))
    for h in sts[_NBLK - _NBUF:]:
        h.wait()


def _toeplitz_rows(g, idx):
    mesh = plsc.VectorSubcoreMesh(core_axis_name="c", subcore_axis_name="s")
    f = functools.partial(
        pl.kernel,
        mesh=mesh,
        out_type=jax.ShapeDtypeStruct((_N, _N), jnp.float32),
        scratch_types=[
            pltpu.VMEM((_NBUF, _BLK, _N), jnp.float32),
            pltpu.VMEM((_NBLK, _BLK), jnp.int32),
            pltpu.SemaphoreType.DMA,
            pltpu.SemaphoreType.DMA,
        ],
    )(_body)
    return f(g, idx)


def kernel(bias_table, query_len, cond_len):
    d = jnp.asarray(query_len, jnp.int32) - jnp.asarray(cond_len, jnp.int32)
    base = jnp.pad(jnp.flip(bias_table), (_N, _N), mode="edge")
    dc = jnp.clip(d, -1900, 1900)
    big = lax.dynamic_slice(base, (_MAXD - dc,), (_EREV + _ROWS,))
    g = jnp.stack(
        [
            jnp.stack(
                [big[127 - 8 * b - gg : 127 - 8 * b - gg + _EREV]
                 for gg in range(_BLK)]
            )
            for b in range(_NBLK)
        ]
    )
    idx = jnp.arange(_N, dtype=jnp.int32).reshape(_NW, _NBLK, _BLK)
    return _toeplitz_rows(g, idx)


# speedup vs baseline: 1.7569x; 1.4803x over previous
import functools

import jax
import jax.numpy as jnp
from jax import lax
from jax.experimental import pallas as pl
from jax.experimental.pallas import tpu as pltpu
from jax.experimental.pallas import tpu_sc as plsc

_MAXD = 2048
_N = 4096
_T = 2 * _MAXD - 1
_EREV = 8192
_NW = 32
_ROWS = _N // _NW
_BLK = 8
_NBLK = _ROWS // _BLK
_NBUF = 3
_GSH = 6144
_NPH = 4


def _body(g_hbm, idx_hbm, out_hbm, gsh, rowbuf, idxv, ldsem, stsem):
    sid = lax.axis_index("s")
    wid = lax.axis_index("c") * 16 + sid
    i0 = wid * _ROWS
    start_min = (_N - _ROWS) - i0

    pltpu.sync_copy(idx_hbm.at[wid], idxv)

    colbase = (1 - lax.axis_index("c")) * (_N // 2)
    half = _NBLK // _NPH

    lds, sts = [], []

    def _scatter(b):
        return pltpu.async_copy(
            rowbuf.at[b % _NBUF], out_hbm.at[idxv.at[b]], stsem
        )

    for phase in range(_NPH):

        @pl.when(sid < half)
        def _load_phase():
            pltpu.sync_copy(
                g_hbm.at[phase * half + sid, :, pl.ds(colbase, _GSH)],
                gsh.at[sid],
            )

        plsc.subcore_barrier()
        for bb in range(half):
            b = phase * half + bb
            if b >= _NBUF:
                sts[b - _NBUF].wait()
            lds.append(
                pltpu.async_copy(
                    gsh.at[bb, :, pl.ds(start_min - colbase, _N)],
                    rowbuf.at[b % _NBUF],
                    ldsem,
                )
            )
            if b >= 1 and len(sts) == b - 1:
                lds[b - 1].wait()
                sts.append(_scatter(b - 1))
        if phase < _NPH - 1:
            last = (phase + 1) * half - 1
            lds[last].wait()
            sts.append(_scatter(last))
            plsc.subcore_barrier()
    lds[_NBLK - 1].wait()
    sts.append(_scatter(_NBLK - 1))
    for h in sts[_NBLK - _NBUF:]:
        h.wait()


def _toeplitz_rows(g, idx):
    mesh = plsc.VectorSubcoreMesh(core_axis_name="c", subcore_axis_name="s")
    f = functools.partial(
        pl.kernel,
        mesh=mesh,
        out_type=jax.ShapeDtypeStruct((_N, _N), jnp.float32),
        scratch_types=[
            pltpu.VMEM_SHARED((_NBLK // _NPH, _BLK, _GSH), jnp.float32),
            pltpu.VMEM((_NBUF, _BLK, _N), jnp.float32),
            pltpu.VMEM((_NBLK, _BLK), jnp.int32),
            pltpu.SemaphoreType.DMA,
            pltpu.SemaphoreType.DMA,
        ],
    )(_body)
    return f(g, idx)


def kernel(bias_table, query_len, cond_len):
    d = jnp.asarray(query_len, jnp.int32) - jnp.asarray(cond_len, jnp.int32)
    base = jnp.pad(jnp.flip(bias_table), (_N, _N), mode="edge")
    dc = jnp.clip(d, -1900, 1900)
    big = lax.dynamic_slice(base, (_MAXD - dc,), (_EREV + _ROWS,))
    g = jnp.stack(
        [
            jnp.stack(
                [big[127 - 8 * b - gg : 127 - 8 * b - gg + _EREV]
                 for gg in range(_BLK)]
            )
            for b in range(_NBLK)
        ]
    )
    idx = jnp.arange(_N, dtype=jnp.int32).reshape(_NW, _NBLK, _BLK)
    return _toeplitz_rows(g, idx)
